# baseline (speedup 1.0000x reference)
import functools

import jax
import jax.numpy as jnp
from jax import lax
from jax.experimental import pallas as pl
from jax.experimental.pallas import tpu as pltpu
from jax.experimental.pallas import tpu_sc as plsc

_N = 10000
_D = 128
_E = 320000
_NC = 2
_NS = 16
_NW = _NC * _NS
_CHUNK = 128
_NCHT = _E // _CHUNK
_NSLOT = (_NCHT + _NW - 1) // _NW
_NPAD = 10240
_RPT = _NPAD // _NS
_RB = 2
_IB = 8


@functools.partial(
    pl.kernel,
    out_type=(
        jax.ShapeDtypeStruct((_NC, _NPAD, _D), jnp.float32),
        jax.ShapeDtypeStruct((_NC, _NPAD), jnp.float32),
    ),
    mesh=plsc.VectorSubcoreMesh(
        core_axis_name="c", subcore_axis_name="s",
        num_cores=_NC, num_subcores=_NS,
    ),
    scratch_types=[
        pltpu.VMEM((_IB, 2, _CHUNK), jnp.int32),
        pltpu.VMEM((_RB, _CHUNK, _D), jnp.float32),
        pltpu.VMEM((_CHUNK,), jnp.float32),
        pltpu.VMEM((_RPT,), jnp.float32),
        pltpu.SemaphoreType.DMA((_IB,)),
        pltpu.SemaphoreType.DMA((_RB,)),
        pltpu.SemaphoreType.DMA((_RB,)),
        pltpu.VMEM_SHARED((_NPAD, _D), jnp.float32),
        pltpu.VMEM_SHARED((_NPAD,), jnp.float32),
    ],
)
def _sc_aggregate(x_hbm, e_hbm, agg_hbm, deg_hbm,
                  ed_idx, rows, ones_v, zde,
                  sem_i, sem_g, sem_s, agg_sh, deg_sh):
    cid = lax.axis_index("c")
    sid = lax.axis_index("s")
    wid = sid * _NC + cid

    zeros16 = jnp.zeros((16,), jnp.float32)
    ones16 = jnp.ones((16,), jnp.float32)

    def _fill_rows(k, carry):
        r = k // (_D // 16)
        c = (k % (_D // 16)) * 16
        rows[0, r, pl.ds(c, 16)] = zeros16
        return carry
    lax.fori_loop(0, _CHUNK * _D // 16, _fill_rows, None)

    def _fill_zde(k, carry):
        zde[pl.ds(k * 16, 16)] = zeros16
        return carry
    lax.fori_loop(0, _RPT // 16, _fill_zde, None)

    def _fill_ones(k, carry):
        ones_v[pl.ds(k * 16, 16)] = ones16
        return carry
    lax.fori_loop(0, _CHUNK // 16, _fill_ones, None)

    r0 = sid * _RPT
    for b in range(_RPT // _CHUNK):
        pltpu.sync_copy(rows.at[0], agg_sh.at[pl.ds(r0 + b * _CHUNK, _CHUNK)])
    pltpu.sync_copy(zde, deg_sh.at[pl.ds(r0, _RPT)])
    plsc.subcore_barrier()

    def start_idx(g, m):
        pltpu.async_copy(e_hbm.at[pl.ds(0, 2), pl.ds(g * _CHUNK, _CHUNK)],
                         ed_idx.at[m], sem_i.at[m])

    def wait_idx(m):
        pltpu.make_async_copy(e_hbm.at[pl.ds(0, 2), pl.ds(0, _CHUNK)],
                              ed_idx.at[m], sem_i.at[m]).wait()

    def start_gather(br, m):
        pltpu.async_copy(x_hbm.at[ed_idx.at[m, 0]], rows.at[br], sem_g.at[br])

    def wait_gather(br):
        pltpu.make_async_copy(x_hbm.at[ed_idx.at[0, 0]], rows.at[br],
                              sem_g.at[br]).wait()

    def start_scatter(br, m):
        pltpu.async_copy(rows.at[br], agg_sh.at[ed_idx.at[m, 1]],
                         sem_s.at[br], add=True)
        pltpu.async_copy(ones_v, deg_sh.at[ed_idx.at[m, 1]], sem_s.at[br],
                         add=True)

    def wait_scatter(br):
        pltpu.make_async_copy(rows.at[br], agg_sh.at[ed_idx.at[0, 1]],
                              sem_s.at[br]).wait()
        pltpu.make_async_copy(ones_v, deg_sh.at[ed_idx.at[0, 1]],
                              sem_s.at[br]).wait()

    for j in range(_IB - 1):
        start_idx(j * _NW + wid, j)
    for j in range(_RB - 1):
        wait_idx(j)
        start_gather(j, j)

    def _slot(t, br, m):
        @pl.when(t >= 1)
        def _():
            wait_scatter((br + _RB - 1) % _RB)

        @pl.when((t + _IB - 1) * _NW + wid < _NCHT)
        def _():
            start_idx((t + _IB - 1) * _NW + wid, (m + _IB - 1) % _IB)

        @pl.when((t + _RB - 1) * _NW + wid < _NCHT)
        def _():
            wait_idx((m + _RB - 1) % _IB)
            start_gather((br + _RB - 1) % _RB, (m + _RB - 1) % _IB)

        @pl.when(t * _NW + wid < _NCHT)
        def _():
            wait_gather(br)
            start_scatter(br, m)

    def _main(i, carry):
        t0 = i * _IB
        for b in range(_IB):
            _slot(t0 + b, b % _RB, b)
        return carry

    _full = (_NSLOT // _IB) * _IB
    lax.fori_loop(0, _NSLOT // _IB, _main, None)
    for t in range(_full, _NSLOT):
        _slot(t, t % _RB, t % _IB)
    @pl.when((_NSLOT - 1) * _NW + wid < _NCHT)
    def _():
        wait_scatter((_NSLOT - 1) % _RB)
    plsc.subcore_barrier()

    pltpu.sync_copy(agg_sh.at[pl.ds(r0, _RPT)], agg_hbm.at[cid, pl.ds(r0, _RPT)])
    pltpu.sync_copy(deg_sh.at[pl.ds(r0, _RPT)], deg_hbm.at[cid, pl.ds(r0, _RPT)])


_BLK = 1000


def _tc_head_body(x_ref, wh_ref, o_ref):
    dn = (((1,), (1,)), ((), ()))
    o_ref[...] = lax.dot_general(x_ref[...], wh_ref[...], dn,
                                 preferred_element_type=jnp.float32)


def _tc_head(x, W_head):
    return pl.pallas_call(
        _tc_head_body,
        grid=(_N // _BLK,),
        in_specs=[
            pl.BlockSpec((_BLK, _D), lambda i: (i, 0)),
            pl.BlockSpec((_D, _D), lambda i: (0, 0)),
        ],
        out_specs=pl.BlockSpec((_BLK, _D), lambda i: (i, 0)),
        out_shape=jax.ShapeDtypeStruct((_N, _D), jnp.float32),
    )(x, W_head)


def _tc_body(h1_ref, agg_ref, deg_ref, wt_ref, g_ref, b_ref, o_ref):
    a = agg_ref[0] + agg_ref[1]
    d = deg_ref[:, 0:1] + deg_ref[:, 1:2]
    a = a * (1.0 / jnp.maximum(d, 1.0))
    dn = (((1,), (1,)), ((), ()))
    h = h1_ref[...] + lax.dot_general(a, wt_ref[...], dn,
                                      preferred_element_type=jnp.float32)
    mu = jnp.mean(h, axis=-1, keepdims=True)
    c = h - mu
    var = jnp.mean(c * c, axis=-1, keepdims=True)
    o_ref[...] = c * lax.rsqrt(var + 1e-5) * g_ref[...] + b_ref[...]


def _tc_combine(h1, agg2, degT, W_tail, gamma2, beta2):
    return pl.pallas_call(
        _tc_body,
        grid=(_N // _BLK,),
        in_specs=[
            pl.BlockSpec((_BLK, _D), lambda i: (i, 0)),
            pl.BlockSpec((_NC, _BLK, _D), lambda i: (0, i, 0)),
            pl.BlockSpec((_BLK, _NC), lambda i: (i, 0)),
            pl.BlockSpec((_D, _D), lambda i: (0, 0)),
            pl.BlockSpec((1, _D), lambda i: (0, 0)),
            pl.BlockSpec((1, _D), lambda i: (0, 0)),
        ],
        out_specs=pl.BlockSpec((_BLK, _D), lambda i: (i, 0)),
        out_shape=jax.ShapeDtypeStruct((_N, _D), jnp.float32),
    )(h1, agg2, degT, W_tail, gamma2, beta2)


def kernel(x, edge_index, W_head, W_tail, gamma, beta):
    agg2, deg2 = _sc_aggregate(x, edge_index)
    h1 = _tc_head(x, W_head)
    return _tc_combine(h1, agg2, deg2.T, W_tail,
                       gamma.reshape(1, _D), beta.reshape(1, _D))


# speedup vs baseline: 1.1007x; 1.1007x over previous
import functools

import jax
import jax.numpy as jnp
from jax import lax
from jax.experimental import pallas as pl
from jax.experimental.pallas import tpu as pltpu
from jax.experimental.pallas import tpu_sc as plsc

_N = 10000
_D = 128
_E = 320000
_NC = 2
_NS = 16
_NW = _NC * _NS
_EPW = _E // _NW
_CHUNK = 80
_NCHUNK = _EPW // _CHUNK
_NPAD = 10240
_RPT = _NPAD // _NS
_RB = 4
_IB = 8


@functools.partial(
    pl.kernel,
    out_type=(
        jax.ShapeDtypeStruct((_NC, _NPAD, _D), jnp.float32),
        jax.ShapeDtypeStruct((_NC, _NPAD), jnp.float32),
    ),
    mesh=plsc.VectorSubcoreMesh(
        core_axis_name="c", subcore_axis_name="s",
        num_cores=_NC, num_subcores=_NS,
    ),
    scratch_types=[
        pltpu.VMEM((_IB, _CHUNK), jnp.int32),
        pltpu.VMEM((_IB, _CHUNK), jnp.int32),
        pltpu.VMEM((_RB, _CHUNK, _D), jnp.float32),
        pltpu.VMEM((_CHUNK,), jnp.float32),
        pltpu.VMEM((_RPT,), jnp.float32),
        pltpu.SemaphoreType.DMA((_IB,)),
        pltpu.SemaphoreType.DMA((_IB,)),
        pltpu.SemaphoreType.DMA((_RB,)),
        pltpu.SemaphoreType.DMA((_RB,)),
        pltpu.VMEM_SHARED((_NPAD, _D), jnp.float32),
        pltpu.VMEM_SHARED((_NPAD,), jnp.float32),
    ],
)
def _sc_aggregate(x_hbm, e_hbm, agg_hbm, deg_hbm,
                  s_idx, d_idx, rows, ones_v, zde,
                  sem_si, sem_di, sem_g, sem_s, agg_sh, deg_sh):
    cid = lax.axis_index("c")
    sid = lax.axis_index("s")
    wid = sid * _NC + cid
    e0 = wid * _EPW

    def start_idx(j, m):
        off = e0 + j * _CHUNK
        pltpu.async_copy(e_hbm.at[pl.ds(off, _CHUNK)], s_idx.at[m],
                         sem_si.at[m])
        pltpu.async_copy(e_hbm.at[pl.ds(_E + off, _CHUNK)], d_idx.at[m],
                         sem_di.at[m])

    def wait_src_idx(m):
        pltpu.make_async_copy(e_hbm.at[pl.ds(e0, _CHUNK)], s_idx.at[m],
                              sem_si.at[m]).wait()

    def wait_dst_idx(m):
        pltpu.make_async_copy(e_hbm.at[pl.ds(e0, _CHUNK)], d_idx.at[m],
                              sem_di.at[m]).wait()

    def start_gather(br, m):
        pltpu.async_copy(x_hbm.at[s_idx.at[m]], rows.at[br], sem_g.at[br])

    def wait_gather(br):
        pltpu.make_async_copy(x_hbm.at[s_idx.at[0]], rows.at[br],
                              sem_g.at[br]).wait()

    def start_scatter(br, m):
        pltpu.async_copy(rows.at[br], agg_sh.at[d_idx.at[m]], sem_s.at[br],
                         add=True)
        pltpu.async_copy(ones_v, deg_sh.at[d_idx.at[m]], sem_s.at[br],
                         add=True)

    def wait_scatter(br):
        pltpu.make_async_copy(rows.at[br], agg_sh.at[d_idx.at[0]],
                              sem_s.at[br]).wait()
        pltpu.make_async_copy(ones_v, deg_sh.at[d_idx.at[0]],
                              sem_s.at[br]).wait()

    for j in range(_IB - 1):
        start_idx(j, j)

    zeros16 = jnp.zeros((16,), jnp.float32)
    ones16 = jnp.ones((16,), jnp.float32)

    def _fill_rows(k, carry):
        r = k // (_D // 16)
        c = (k % (_D // 16)) * 16
        rows[0, r, pl.ds(c, 16)] = zeros16
        return carry
    lax.fori_loop(0, _CHUNK * _D // 16, _fill_rows, None)

    def _fill_zde(k, carry):
        zde[pl.ds(k * 16, 16)] = zeros16
        return carry
    lax.fori_loop(0, _RPT // 16, _fill_zde, None)

    def _fill_ones(k, carry):
        ones_v[pl.ds(k * 16, 16)] = ones16
        return carry
    lax.fori_loop(0, _CHUNK // 16, _fill_ones, None)

    r0 = sid * _RPT
    for b in range(_RPT // _CHUNK):
        pltpu.sync_copy(rows.at[0], agg_sh.at[pl.ds(r0 + b * _CHUNK, _CHUNK)])
    pltpu.sync_copy(zde, deg_sh.at[pl.ds(r0, _RPT)])
    plsc.subcore_barrier()

    for j in range(_RB - 1):
        wait_src_idx(j)
        start_gather(j, j)

    def _slot(j, br, m):
        @pl.when(j >= 1)
        def _():
            wait_scatter((br + _RB - 1) % _RB)

        @pl.when(j + _IB - 1 < _NCHUNK)
        def _():
            start_idx(j + _IB - 1, (m + _IB - 1) % _IB)

        @pl.when(j + _RB - 1 < _NCHUNK)
        def _():
            wait_src_idx((m + _RB - 1) % _IB)
            start_gather((br + _RB - 1) % _RB, (m + _RB - 1) % _IB)

        wait_gather(br)
        wait_dst_idx(m)
        start_scatter(br, m)

    def _main(i, carry):
        j0 = i * _IB
        for b in range(_IB):
            _slot(j0 + b, b % _RB, b)
        return carry

    _full = (_NCHUNK // _IB) * _IB
    lax.fori_loop(0, _NCHUNK // _IB, _main, None)
    for j in range(_full, _NCHUNK):
        _slot(j, j % _RB, j % _IB)
    wait_scatter((_NCHUNK - 1) % _RB)
    plsc.subcore_barrier()

    pltpu.sync_copy(agg_sh.at[pl.ds(r0, _RPT)], agg_hbm.at[cid, pl.ds(r0, _RPT)])
    pltpu.sync_copy(deg_sh.at[pl.ds(r0, _RPT)], deg_hbm.at[cid, pl.ds(r0, _RPT)])


_BLK = 1000


def _tc_head_body(x_ref, wh_ref, o_ref):
    dn = (((1,), (1,)), ((), ()))
    o_ref[...] = lax.dot_general(x_ref[...], wh_ref[...], dn,
                                 preferred_element_type=jnp.float32)


def _tc_head(x, W_head):
    return pl.pallas_call(
        _tc_head_body,
        grid=(_N // _BLK,),
        in_specs=[
            pl.BlockSpec((_BLK, _D), lambda i: (i, 0)),
            pl.BlockSpec((_D, _D), lambda i: (0, 0)),
        ],
        out_specs=pl.BlockSpec((_BLK, _D), lambda i: (i, 0)),
        out_shape=jax.ShapeDtypeStruct((_N, _D), jnp.float32),
    )(x, W_head)


def _tc_body(h1_ref, agg_ref, deg_ref, wt_ref, g_ref, b_ref, o_ref):
    a = agg_ref[0] + agg_ref[1]
    d = deg_ref[:, 0:1] + deg_ref[:, 1:2]
    a = a * (1.0 / jnp.maximum(d, 1.0))
    dn = (((1,), (1,)), ((), ()))
    h = h1_ref[...] + lax.dot_general(a, wt_ref[...], dn,
                                      preferred_element_type=jnp.float32)
    mu = jnp.mean(h, axis=-1, keepdims=True)
    c = h - mu
    var = jnp.mean(c * c, axis=-1, keepdims=True)
    o_ref[...] = c * lax.rsqrt(var + 1e-5) * g_ref[...] + b_ref[...]


def _tc_combine(h1, agg2, degT, W_tail, gamma2, beta2):
    return pl.pallas_call(
        _tc_body,
        grid=(_N // _BLK,),
        in_specs=[
            pl.BlockSpec((_BLK, _D), lambda i: (i, 0)),
            pl.BlockSpec((_NC, _BLK, _D), lambda i: (0, i, 0)),
            pl.BlockSpec((_BLK, _NC), lambda i: (i, 0)),
            pl.BlockSpec((_D, _D), lambda i: (0, 0)),
            pl.BlockSpec((1, _D), lambda i: (0, 0)),
            pl.BlockSpec((1, _D), lambda i: (0, 0)),
        ],
        out_specs=pl.BlockSpec((_BLK, _D), lambda i: (i, 0)),
        out_shape=jax.ShapeDtypeStruct((_N, _D), jnp.float32),
    )(h1, agg2, degT, W_tail, gamma2, beta2)


def kernel(x, edge_index, W_head, W_tail, gamma, beta):
    agg2, deg2 = _sc_aggregate(x, edge_index.reshape(2 * _E))
    h1 = _tc_head(x, W_head)
    return _tc_combine(h1, agg2, deg2.T, W_tail,
                       gamma.reshape(1, _D), beta.reshape(1, _D))


# speedup vs baseline: 1.1212x; 1.0186x over previous
import functools

import jax
import jax.numpy as jnp
from jax import lax
from jax.experimental import pallas as pl
from jax.experimental.pallas import tpu as pltpu
from jax.experimental.pallas import tpu_sc as plsc

_N = 10000
_D = 128
_E = 320000
_NC = 2
_NS = 16
_NW = _NC * _NS
_EPW = _E // _NW
_CHUNK = 80
_NCHUNK = _EPW // _CHUNK
_NPAD = 10240
_RPT = _NPAD // _NS
_RB = 4
_IB = 8


@functools.partial(
    pl.kernel,
    out_type=(
        jax.ShapeDtypeStruct((_NC, _NPAD, _D), jnp.float32),
        jax.ShapeDtypeStruct((_NC, _NPAD), jnp.float32),
    ),
    mesh=plsc.VectorSubcoreMesh(
        core_axis_name="c", subcore_axis_name="s",
        num_cores=_NC, num_subcores=_NS,
    ),
    scratch_types=[
        pltpu.VMEM((_IB, _CHUNK), jnp.int32),
        pltpu.VMEM((_IB, _CHUNK), jnp.int32),
        pltpu.VMEM((_RB, _CHUNK, _D), jnp.float32),
        pltpu.VMEM((_CHUNK,), jnp.float32),
        pltpu.VMEM((_RPT,), jnp.float32),
        pltpu.SemaphoreType.DMA((_IB,)),
        pltpu.SemaphoreType.DMA((_IB,)),
        pltpu.SemaphoreType.DMA((_RB,)),
        pltpu.SemaphoreType.DMA((_RB,)),
        pltpu.VMEM_SHARED((_NPAD, _D), jnp.float32),
        pltpu.VMEM_SHARED((_NPAD,), jnp.float32),
    ],
)
def _sc_aggregate(x_hbm, e_hbm, agg_hbm, deg_hbm,
                  s_idx, d_idx, rows, ones_v, zde,
                  sem_si, sem_di, sem_g, sem_s, agg_sh, deg_sh):
    cid = lax.axis_index("c")
    sid = lax.axis_index("s")
    wid = sid * _NC + cid
    e0 = wid * _EPW

    def start_idx(j, m):
        off = e0 + j * _CHUNK
        pltpu.async_copy(e_hbm.at[pl.ds(off, _CHUNK)], s_idx.at[m],
                         sem_si.at[m])
        pltpu.async_copy(e_hbm.at[pl.ds(_E + off, _CHUNK)], d_idx.at[m],
                         sem_di.at[m])

    def wait_src_idx(m):
        pltpu.make_async_copy(e_hbm.at[pl.ds(e0, _CHUNK)], s_idx.at[m],
                              sem_si.at[m]).wait()

    def wait_dst_idx(m):
        pltpu.make_async_copy(e_hbm.at[pl.ds(e0, _CHUNK)], d_idx.at[m],
                              sem_di.at[m]).wait()

    def start_gather(br, m):
        pltpu.async_copy(x_hbm.at[s_idx.at[m]], rows.at[br], sem_g.at[br])

    def wait_gather(br):
        pltpu.make_async_copy(x_hbm.at[s_idx.at[0]], rows.at[br],
                              sem_g.at[br]).wait()

    def start_scatter(br, m):
        pltpu.async_copy(rows.at[br], agg_sh.at[d_idx.at[m]], sem_s.at[br],
                         add=True)
        pltpu.async_copy(ones_v, deg_sh.at[d_idx.at[m]], sem_s.at[br],
                         add=True)

    def wait_scatter(br):
        pltpu.make_async_copy(rows.at[br], agg_sh.at[d_idx.at[0]],
                              sem_s.at[br]).wait()
        pltpu.make_async_copy(ones_v, deg_sh.at[d_idx.at[0]],
                              sem_s.at[br]).wait()

    for j in range(_IB - 1):
        start_idx(j, j)

    zeros16 = jnp.zeros((16,), jnp.float32)
    ones16 = jnp.ones((16,), jnp.float32)

    def _fill_rows(k, carry):
        r = k // (_D // 16)
        c = (k % (_D // 16)) * 16
        rows[0, r, pl.ds(c, 16)] = zeros16
        return carry
    lax.fori_loop(0, _CHUNK * _D // 16, _fill_rows, None)

    def _fill_zde(k, carry):
        zde[pl.ds(k * 16, 16)] = zeros16
        return carry
    lax.fori_loop(0, _RPT // 16, _fill_zde, None)

    def _fill_ones(k, carry):
        ones_v[pl.ds(k * 16, 16)] = ones16
        return carry
    lax.fori_loop(0, _CHUNK // 16, _fill_ones, None)

    r0 = sid * _RPT
    for b in range(_RPT // _CHUNK):
        pltpu.sync_copy(rows.at[0], agg_sh.at[pl.ds(r0 + b * _CHUNK, _CHUNK)])
    pltpu.sync_copy(zde, deg_sh.at[pl.ds(r0, _RPT)])
    plsc.subcore_barrier()

    for j in range(_RB - 1):
        wait_src_idx(j)
        start_gather(j, j)

    def _slot(j, br, m):
        @pl.when(j >= 1)
        def _():
            wait_scatter((br + _RB - 1) % _RB)

        @pl.when(j + _IB - 1 < _NCHUNK)
        def _():
            start_idx(j + _IB - 1, (m + _IB - 1) % _IB)

        @pl.when(j + _RB - 1 < _NCHUNK)
        def _():
            wait_src_idx((m + _RB - 1) % _IB)
            start_gather((br + _RB - 1) % _RB, (m + _RB - 1) % _IB)

        wait_gather(br)
        wait_dst_idx(m)
        start_scatter(br, m)

    def _main(i, carry):
        j0 = i * _IB
        for b in range(_IB):
            _slot(j0 + b, b % _RB, b)
        return carry

    _full = (_NCHUNK // _IB) * _IB
    lax.fori_loop(0, _NCHUNK // _IB, _main, None)
    for j in range(_full, _NCHUNK):
        _slot(j, j % _RB, j % _IB)
    wait_scatter((_NCHUNK - 1) % _RB)
    plsc.subcore_barrier()

    pltpu.sync_copy(agg_sh.at[pl.ds(r0, _RPT)], agg_hbm.at[cid, pl.ds(r0, _RPT)])
    pltpu.sync_copy(deg_sh.at[pl.ds(r0, _RPT)], deg_hbm.at[cid, pl.ds(r0, _RPT)])


_BLK = 2000


def _tc_head_body(x_ref, wh_ref, o_ref):
    dn = (((1,), (1,)), ((), ()))
    o_ref[...] = lax.dot_general(x_ref[...], wh_ref[...], dn,
                                 preferred_element_type=jnp.float32)


def _tc_head(x, W_head):
    return pl.pallas_call(
        _tc_head_body,
        grid=(_N // _BLK,),
        in_specs=[
            pl.BlockSpec((_BLK, _D), lambda i: (i, 0)),
            pl.BlockSpec((_D, _D), lambda i: (0, 0)),
        ],
        out_specs=pl.BlockSpec((_BLK, _D), lambda i: (i, 0)),
        out_shape=jax.ShapeDtypeStruct((_N, _D), jnp.float32),
    )(x, W_head)


def _tc_body(h1_ref, agg_ref, deg_ref, wt_ref, g_ref, b_ref, o_ref):
    a = agg_ref[0] + agg_ref[1]
    d = deg_ref[:, 0:1] + deg_ref[:, 1:2]
    a = a * (1.0 / jnp.maximum(d, 1.0))
    dn = (((1,), (1,)), ((), ()))
    h = h1_ref[...] + lax.dot_general(a, wt_ref[...], dn,
                                      preferred_element_type=jnp.float32)
    mu = jnp.mean(h, axis=-1, keepdims=True)
    c = h - mu
    var = jnp.mean(c * c, axis=-1, keepdims=True)
    o_ref[...] = c * lax.rsqrt(var + 1e-5) * g_ref[...] + b_ref[...]


def _tc_combine(h1, agg2, degT, W_tail, gamma2, beta2):
    return pl.pallas_call(
        _tc_body,
        grid=(_N // _BLK,),
        in_specs=[
            pl.BlockSpec((_BLK, _D), lambda i: (i, 0)),
            pl.BlockSpec((_NC, _BLK, _D), lambda i: (0, i, 0)),
            pl.BlockSpec((_BLK, _NC), lambda i: (i, 0)),
            pl.BlockSpec((_D, _D), lambda i: (0, 0)),
            pl.BlockSpec((1, _D), lambda i: (0, 0)),
            pl.BlockSpec((1, _D), lambda i: (0, 0)),
        ],
        out_specs=pl.BlockSpec((_BLK, _D), lambda i: (i, 0)),
        out_shape=jax.ShapeDtypeStruct((_N, _D), jnp.float32),
    )(h1, agg2, degT, W_tail, gamma2, beta2)


def kernel(x, edge_index, W_head, W_tail, gamma, beta):
    agg2, deg2 = _sc_aggregate(x, edge_index.reshape(2 * _E))
    h1 = _tc_head(x, W_head)
    return _tc_combine(h1, agg2, deg2.T, W_tail,
                       gamma.reshape(1, _D), beta.reshape(1, _D))


# speedup vs baseline: 1.1279x; 1.0060x over previous
import functools

import jax
import jax.numpy as jnp
from jax import lax
from jax.experimental import pallas as pl
from jax.experimental.pallas import tpu as pltpu
from jax.experimental.pallas import tpu_sc as plsc

_N = 10000
_D = 128
_E = 320000
_NC = 2
_NS = 16
_NW = _NC * _NS
_EPW = _E // _NW
_CHUNK = 80
_NCHUNK = _EPW // _CHUNK
_NPAD = 10240
_RPT = _NPAD // _NS
_RB = 4
_IB = 8


@functools.partial(
    pl.kernel,
    out_type=(
        jax.ShapeDtypeStruct((_NC, _NPAD, _D), jnp.float32),
        jax.ShapeDtypeStruct((_NC, _NPAD), jnp.float32),
    ),
    mesh=plsc.VectorSubcoreMesh(
        core_axis_name="c", subcore_axis_name="s",
        num_cores=_NC, num_subcores=_NS,
    ),
    scratch_types=[
        pltpu.VMEM((_IB, _CHUNK), jnp.int32),
        pltpu.VMEM((_IB, _CHUNK), jnp.int32),
        pltpu.VMEM((_RB, _CHUNK, _D), jnp.float32),
        pltpu.VMEM((_CHUNK,), jnp.float32),
        pltpu.VMEM((_RPT,), jnp.float32),
        pltpu.SemaphoreType.DMA((_IB,)),
        pltpu.SemaphoreType.DMA((_IB,)),
        pltpu.SemaphoreType.DMA((_RB,)),
        pltpu.SemaphoreType.DMA((_RB,)),
        pltpu.VMEM_SHARED((_NPAD, _D), jnp.float32),
        pltpu.VMEM_SHARED((_NPAD,), jnp.float32),
    ],
)
def _sc_aggregate(x_hbm, e_hbm, agg_hbm, deg_hbm,
                  s_idx, d_idx, rows, ones_v, zde,
                  sem_si, sem_di, sem_g, sem_s, agg_sh, deg_sh):
    cid = lax.axis_index("c")
    sid = lax.axis_index("s")
    wid = sid * _NC + cid
    e0 = wid * _EPW

    def start_idx(j, m):
        off = e0 + j * _CHUNK
        pltpu.async_copy(e_hbm.at[pl.ds(off, _CHUNK)], s_idx.at[m],
                         sem_si.at[m])
        pltpu.async_copy(e_hbm.at[pl.ds(_E + off, _CHUNK)], d_idx.at[m],
                         sem_di.at[m])

    def wait_src_idx(m):
        pltpu.make_async_copy(e_hbm.at[pl.ds(e0, _CHUNK)], s_idx.at[m],
                              sem_si.at[m]).wait()

    def wait_dst_idx(m):
        pltpu.make_async_copy(e_hbm.at[pl.ds(e0, _CHUNK)], d_idx.at[m],
                              sem_di.at[m]).wait()

    def start_gather(br, m):
        pltpu.async_copy(x_hbm.at[s_idx.at[m]], rows.at[br], sem_g.at[br])

    def wait_gather(br):
        pltpu.make_async_copy(x_hbm.at[s_idx.at[0]], rows.at[br],
                              sem_g.at[br]).wait()

    def start_scatter(br, m):
        pltpu.async_copy(rows.at[br], agg_sh.at[d_idx.at[m]], sem_s.at[br],
                         add=True)
        pltpu.async_copy(ones_v, deg_sh.at[d_idx.at[m]], sem_s.at[br],
                         add=True)

    def wait_scatter(br):
        pltpu.make_async_copy(rows.at[br], agg_sh.at[d_idx.at[0]],
                              sem_s.at[br]).wait()
        pltpu.make_async_copy(ones_v, deg_sh.at[d_idx.at[0]],
                              sem_s.at[br]).wait()

    for j in range(_IB - 1):
        start_idx(j, j)

    zeros16 = jnp.zeros((16,), jnp.float32)
    ones16 = jnp.ones((16,), jnp.float32)

    def _fill_rows(k, carry):
        r = k // (_D // 16)
        c = (k % (_D // 16)) * 16
        rows[0, r, pl.ds(c, 16)] = zeros16
        return carry
    lax.fori_loop(0, _CHUNK * _D // 16, _fill_rows, None)

    def _fill_zde(k, carry):
        zde[pl.ds(k * 16, 16)] = zeros16
        return carry
    lax.fori_loop(0, _RPT // 16, _fill_zde, None)

    def _fill_ones(k, carry):
        ones_v[pl.ds(k * 16, 16)] = ones16
        return carry
    lax.fori_loop(0, _CHUNK // 16, _fill_ones, None)

    r0 = sid * _RPT
    for b in range(_RPT // _CHUNK):
        pltpu.sync_copy(rows.at[0], agg_sh.at[pl.ds(r0 + b * _CHUNK, _CHUNK)])
    pltpu.sync_copy(zde, deg_sh.at[pl.ds(r0, _RPT)])
    plsc.subcore_barrier()

    for j in range(_RB - 1):
        wait_src_idx(j)
        start_gather(j, j)

    def _slot(j, br, m):
        @pl.when(j >= 1)
        def _():
            wait_scatter((br + _RB - 1) % _RB)

        @pl.when(j + _IB - 1 < _NCHUNK)
        def _():
            start_idx(j + _IB - 1, (m + _IB - 1) % _IB)

        @pl.when(j + _RB - 1 < _NCHUNK)
        def _():
            wait_src_idx((m + _RB - 1) % _IB)
            start_gather((br + _RB - 1) % _RB, (m + _RB - 1) % _IB)

        wait_gather(br)
        wait_dst_idx(m)
        start_scatter(br, m)

    def _main(i, carry):
        j0 = i * _IB
        for b in range(_IB):
            _slot(j0 + b, b % _RB, b)
        return carry

    _full = (_NCHUNK // _IB) * _IB
    lax.fori_loop(0, _NCHUNK // _IB, _main, None)
    for j in range(_full, _NCHUNK):
        _slot(j, j % _RB, j % _IB)
    wait_scatter((_NCHUNK - 1) % _RB)
    plsc.subcore_barrier()

    pltpu.sync_copy(agg_sh.at[pl.ds(r0, _RPT)], agg_hbm.at[cid, pl.ds(r0, _RPT)])
    pltpu.sync_copy(deg_sh.at[pl.ds(r0, _RPT)], deg_hbm.at[cid, pl.ds(r0, _RPT)])


_BLK = 5000


def _tc_head_body(x_ref, wh_ref, o_ref):
    dn = (((1,), (1,)), ((), ()))
    o_ref[...] = lax.dot_general(x_ref[...], wh_ref[...], dn,
                                 preferred_element_type=jnp.float32)


def _tc_head(x, W_head):
    return pl.pallas_call(
        _tc_head_body,
        grid=(_N // _BLK,),
        in_specs=[
            pl.BlockSpec((_BLK, _D), lambda i: (i, 0)),
            pl.BlockSpec((_D, _D), lambda i: (0, 0)),
        ],
        out_specs=pl.BlockSpec((_BLK, _D), lambda i: (i, 0)),
        out_shape=jax.ShapeDtypeStruct((_N, _D), jnp.float32),
    )(x, W_head)


def _tc_body(h1_ref, agg_ref, deg_ref, wt_ref, g_ref, b_ref, o_ref):
    a = agg_ref[0] + agg_ref[1]
    d = deg_ref[:, 0:1] + deg_ref[:, 1:2]
    a = a * (1.0 / jnp.maximum(d, 1.0))
    dn = (((1,), (1,)), ((), ()))
    h = h1_ref[...] + lax.dot_general(a, wt_ref[...], dn,
                                      preferred_element_type=jnp.float32)
    mu = jnp.mean(h, axis=-1, keepdims=True)
    c = h - mu
    var = jnp.mean(c * c, axis=-1, keepdims=True)
    o_ref[...] = c * lax.rsqrt(var + 1e-5) * g_ref[...] + b_ref[...]


def _tc_combine(h1, agg2, degT, W_tail, gamma2, beta2):
    return pl.pallas_call(
        _tc_body,
        grid=(_N // _BLK,),
        in_specs=[
            pl.BlockSpec((_BLK, _D), lambda i: (i, 0)),
            pl.BlockSpec((_NC, _BLK, _D), lambda i: (0, i, 0)),
            pl.BlockSpec((_BLK, _NC), lambda i: (i, 0)),
            pl.BlockSpec((_D, _D), lambda i: (0, 0)),
            pl.BlockSpec((1, _D), lambda i: (0, 0)),
            pl.BlockSpec((1, _D), lambda i: (0, 0)),
        ],
        out_specs=pl.BlockSpec((_BLK, _D), lambda i: (i, 0)),
        out_shape=jax.ShapeDtypeStruct((_N, _D), jnp.float32),
    )(h1, agg2, degT, W_tail, gamma2, beta2)


def kernel(x, edge_index, W_head, W_tail, gamma, beta):
    agg2, deg2 = _sc_aggregate(x, edge_index.reshape(2 * _E))
    h1 = _tc_head(x, W_head)
    return _tc_combine(h1, agg2, deg2.T, W_tail,
                       gamma.reshape(1, _D), beta.reshape(1, _D))
